# baseline (speedup 1.0000x reference)
import functools

import jax
import jax.numpy as jnp
from jax import lax
from jax.experimental import pallas as pl
from jax.experimental.pallas import tpu as pltpu
from jax.experimental.pallas import tpu_sc as plsc

_N = 10000
_E = 320000
_D = 128
_H = 128

_NC = 2
_NS = 16
_NW = _NC * _NS

_K = 128
_NCHUNK = 84
_EPT = _K * _NCHUNK
_EPAD = _EPT * _NW

_NP = 10112
_RPT = _NP // _NS
_ZROWS = 128

_DEGP = 10240
_DPT = _DEGP // _NS

_ROWBLK = 1000
_GRID = _N // _ROWBLK

_mesh = plsc.VectorSubcoreMesh(core_axis_name="c", subcore_axis_name="s")


@functools.partial(
    pl.kernel,
    out_type=jax.ShapeDtypeStruct((_NC, _DEGP), jnp.float32),
    mesh=_mesh,
    scratch_types=[
        pltpu.VMEM_SHARED((_DEGP,), jnp.float32),
        pltpu.VMEM((_K,), jnp.int32),
        pltpu.VMEM((_K,), jnp.float32),
        pltpu.VMEM((_DPT,), jnp.float32),
    ],
)
def _sc_degree(c_hbm, ew_hbm, out_hbm, deg_sh, c_v, ew_v, zbuf):
    cid = lax.axis_index("c")
    sid = lax.axis_index("s")
    wid = sid * _NC + cid

    for i in range(_DPT // 16):
        zbuf[pl.ds(i * 16, 16)] = jnp.zeros((16,), jnp.float32)
    pltpu.sync_copy(zbuf, deg_sh.at[pl.ds(sid * _DPT, _DPT)])
    plsc.subcore_barrier()

    base = wid * _EPT

    def chunk(i, carry):
        off = base + i * _K
        pltpu.sync_copy(c_hbm.at[pl.ds(off, _K)], c_v)
        pltpu.sync_copy(ew_hbm.at[pl.ds(off, _K)], ew_v)
        pltpu.sync_copy(ew_v, deg_sh.at[c_v], add=True)
        return carry

    lax.fori_loop(0, _NCHUNK, chunk, 0)
    plsc.subcore_barrier()
    pltpu.sync_copy(deg_sh.at[pl.ds(sid * _DPT, _DPT)],
                    out_hbm.at[cid, pl.ds(sid * _DPT, _DPT)])


@functools.partial(
    pl.kernel,
    out_type=jax.ShapeDtypeStruct((_NC, _NP, _H), jnp.float32),
    mesh=_mesh,
    scratch_types=[
        pltpu.VMEM_SHARED((_NP, _H), jnp.float32),
        pltpu.VMEM((_K, _H), jnp.float32),
        pltpu.VMEM((_K, _H), jnp.float32),
        pltpu.VMEM((_K, _H), jnp.float32),
        pltpu.VMEM((_K,), jnp.int32),
        pltpu.VMEM((_K,), jnp.int32),
        pltpu.VMEM((_K,), jnp.int32),
        pltpu.VMEM((_K,), jnp.int32),
        pltpu.VMEM((_K,), jnp.int32),
        pltpu.VMEM((_K,), jnp.float32),
        pltpu.VMEM((_K,), jnp.float32),
        pltpu.SemaphoreType.DMA,
        pltpu.SemaphoreType.DMA,
        pltpu.SemaphoreType.DMA,
        pltpu.SemaphoreType.DMA,
        pltpu.SemaphoreType.DMA,
    ],
)
def _sc_message(g_hbm, r_hbm, c_hbm, ew_hbm, out_hbm,
                acc, rows0, rows1, rows2, r0, r1, r2, c0, c1, w0, w1,
                gsem0, gsem1, gsem2, isem0, isem1):
    cid = lax.axis_index("c")
    sid = lax.axis_index("s")
    wid = sid * _NC + cid
    rows_b = (rows0, rows1, rows2)
    r_b = (r0, r1, r2)
    c_b = (c0, c1)
    w_b = (w0, w1)
    gsem_b = (gsem0, gsem1, gsem2)
    isem_b = (isem0, isem1)

    def zrow(i, carry):
        for q in range(_H // 16):
            rows0[i, pl.ds(q * 16, 16)] = jnp.zeros((16,), jnp.float32)
        return carry

    lax.fori_loop(0, _ZROWS, zrow, 0)
    for t in range(4):
        pltpu.sync_copy(rows0, acc.at[pl.ds(sid * _RPT + t * _ZROWS, _ZROWS)])
    pltpu.sync_copy(rows0.at[pl.ds(0, _RPT - 4 * _ZROWS)],
                    acc.at[pl.ds(sid * _RPT + 4 * _ZROWS, _RPT - 4 * _ZROWS)])
    plsc.subcore_barrier()

    base = wid * _EPT

    def idx_descs(i, i3, i2):
        off = base + i * _K
        sem = isem_b[i2]
        return (
            pltpu.make_async_copy(r_hbm.at[pl.ds(off, _K)], r_b[i3], sem),
            pltpu.make_async_copy(c_hbm.at[pl.ds(off, _K)], c_b[i2], sem),
            pltpu.make_async_copy(ew_hbm.at[pl.ds(off, _K)], w_b[i2], sem),
        )

    for d in idx_descs(0, 0, 0):
        d.start()
    for d in idx_descs(0, 0, 0):
        d.wait()
    pltpu.async_copy(g_hbm.at[r0], rows0, gsem0)
    for d in idx_descs(1, 1, 1):
        d.start()

    def chunk6(io, carry):
        for u in range(6):
            i = io * 6 + u
            b3 = u % 3
            b2 = u % 2
            n3 = (u + 1) % 3
            n2 = (u + 1) % 2

            def _advance():
                for d in idx_descs(i + 1, n3, n2):
                    d.wait()
                pltpu.async_copy(g_hbm.at[r_b[n3]], rows_b[n3], gsem_b[n3])

            if u < 5:
                _advance()
            else:
                pl.when(io < _NCHUNK // 6 - 1)(_advance)

            pltpu.make_async_copy(g_hbm.at[r_b[b3]], rows_b[b3],
                                  gsem_b[b3]).wait()

            def scale16(jo, inner):
                j0 = jo * 16
                wv = w_b[b2][pl.ds(j0, 16)]
                for jj in range(16):
                    w = jnp.full((16,), wv[jj], jnp.float32)
                    for q in range(_H // 16):
                        rows_b[b3][j0 + jj, pl.ds(q * 16, 16)] = (
                            rows_b[b3][j0 + jj, pl.ds(q * 16, 16)] * w)
                return inner

            lax.fori_loop(0, _K // 16, scale16, 0)

            pltpu.sync_copy(rows_b[b3], acc.at[c_b[b2]], add=True)

            @pl.when(i + 2 < _NCHUNK)
            def _next_idx():
                for d in idx_descs(i + 2, (u + 2) % 3, b2):
                    d.start()
        return carry

    lax.fori_loop(0, _NCHUNK // 6, chunk6, 0)
    plsc.subcore_barrier()
    for t in range(4):
        o = sid * _RPT + t * _ZROWS
        pltpu.sync_copy(acc.at[pl.ds(o, _ZROWS)],
                        out_hbm.at[cid, pl.ds(o, _ZROWS)])
    o = sid * _RPT + 4 * _ZROWS
    pltpu.sync_copy(acc.at[pl.ds(o, _RPT - 4 * _ZROWS)],
                    out_hbm.at[cid, pl.ds(o, _RPT - 4 * _ZROWS)])


def _tc1_body(d0_ref, d1_ref, x_ref, w1_ref, dis_ref, g1_ref):
    deg = d0_ref[...] + d1_ref[...] + 1.0
    dis = jnp.where(deg > 0.0, lax.rsqrt(deg), 0.0)
    dis_ref[...] = dis
    h = jnp.dot(x_ref[...], w1_ref[...], preferred_element_type=jnp.float32)
    g1_ref[...] = h * dis


def _tc2_body(a0_ref, a1_ref, g1_ref, dis_ref, b1_ref, w2_ref,
              h1_ref, g2_ref):
    dis = dis_ref[...]
    pre = (a0_ref[...] + a1_ref[...] + g1_ref[...]) * dis + b1_ref[...]
    h1 = jnp.maximum(pre, 0.0)
    h1_ref[...] = h1
    g2_ref[...] = jnp.dot(h1, w2_ref[...],
                          preferred_element_type=jnp.float32) * dis


def _tc3_body(a0_ref, a1_ref, g2_ref, dis_ref, b2_ref, h2_ref):
    pre = ((a0_ref[...] + a1_ref[...] + g2_ref[...]) * dis_ref[...]
           + b2_ref[...])
    h2_ref[...] = jnp.maximum(pre, 0.0)


def _row_blk(shape_cols):
    return pl.BlockSpec((_ROWBLK, shape_cols), lambda i: (i, 0))


def _full_blk(rows, cols):
    return pl.BlockSpec((rows, cols), lambda i: (0, 0))


_tc1 = pl.pallas_call(
    _tc1_body,
    grid=(_GRID,),
    in_specs=[
        _row_blk(1), _row_blk(1), _row_blk(_D), _full_blk(_D, _H),
    ],
    out_specs=[_row_blk(1), _row_blk(_H)],
    out_shape=[
        jax.ShapeDtypeStruct((_N, 1), jnp.float32),
        jax.ShapeDtypeStruct((_N, _H), jnp.float32),
    ],
)

_tc2 = pl.pallas_call(
    _tc2_body,
    grid=(_GRID,),
    in_specs=[
        _row_blk(_H), _row_blk(_H), _row_blk(_H), _row_blk(1),
        _full_blk(1, _H), _full_blk(_H, _H),
    ],
    out_specs=[_row_blk(_H), _row_blk(_H)],
    out_shape=[
        jax.ShapeDtypeStruct((_N, _H), jnp.float32),
        jax.ShapeDtypeStruct((_N, _H), jnp.float32),
    ],
)

_tc3 = pl.pallas_call(
    _tc3_body,
    grid=(_GRID,),
    in_specs=[
        _row_blk(_H), _row_blk(_H), _row_blk(_H), _row_blk(1),
        _full_blk(1, _H),
    ],
    out_specs=_row_blk(_H),
    out_shape=jax.ShapeDtypeStruct((_N, _H), jnp.float32),
)


@jax.jit
def kernel(x, edge_index, edge_weights, W1, b1, W2, b2):
    row = edge_index[0]
    col = edge_index[1]
    pad = _EPAD - _E
    row_p = jnp.concatenate([row, jnp.zeros((pad,), jnp.int32)])
    col_p = jnp.concatenate([col, jnp.zeros((pad,), jnp.int32)])
    ew_p = jnp.concatenate([edge_weights, jnp.zeros((pad,), jnp.float32)])

    deg_parts = _sc_degree(col_p, ew_p)
    d0 = deg_parts[0, :_N].reshape(_N, 1)
    d1 = deg_parts[1, :_N].reshape(_N, 1)

    dis, g1 = _tc1(d0, d1, x, W1)

    acc1 = _sc_message(g1, row_p, col_p, ew_p)
    h1, g2 = _tc2(acc1[0, :_N], acc1[1, :_N], g1, dis,
                  b1.reshape(1, _H), W2)

    acc2 = _sc_message(g2, row_p, col_p, ew_p)
    h2 = _tc3(acc2[0, :_N], acc2[1, :_N], g2, dis, b2.reshape(1, _H))

    return jnp.concatenate([h1, h2], axis=-1)


# speedup vs baseline: 2.2584x; 2.2584x over previous
import functools

import jax
import jax.numpy as jnp
from jax import lax
from jax.experimental import pallas as pl
from jax.experimental.pallas import tpu as pltpu
from jax.experimental.pallas import tpu_sc as plsc

_N = 10000
_E = 320000
_D = 128
_H = 128

_NC = 2
_NS = 16
_NW = _NC * _NS

_K = 48
_NCHUNK = 224
_DK = 128
_DCH = 84
_EPT = _K * _NCHUNK
_EPAD = _EPT * _NW

_NP = 10112
_RPT = _NP // _NS
_ZROWS = 128

_DEGP = 10240
_DPT = _DEGP // _NS

_ROWBLK = 1000
_GRID = _N // _ROWBLK

_mesh = plsc.VectorSubcoreMesh(core_axis_name="c", subcore_axis_name="s")


@functools.partial(
    pl.kernel,
    out_type=jax.ShapeDtypeStruct((_NC, _DEGP), jnp.float32),
    mesh=_mesh,
    scratch_types=[
        pltpu.VMEM_SHARED((_DEGP,), jnp.float32),
        pltpu.VMEM((_DK,), jnp.int32),
        pltpu.VMEM((_DK,), jnp.float32),
        pltpu.VMEM((_DPT,), jnp.float32),
    ],
)
def _sc_degree(c_hbm, ew_hbm, out_hbm, deg_sh, c_v, ew_v, zbuf):
    cid = lax.axis_index("c")
    sid = lax.axis_index("s")
    wid = sid * _NC + cid

    for i in range(_DPT // 16):
        zbuf[pl.ds(i * 16, 16)] = jnp.zeros((16,), jnp.float32)
    pltpu.sync_copy(zbuf, deg_sh.at[pl.ds(sid * _DPT, _DPT)])
    plsc.subcore_barrier()

    base = wid * _EPT

    def chunk(i, carry):
        off = base + i * _DK
        pltpu.sync_copy(c_hbm.at[pl.ds(off, _DK)], c_v)
        pltpu.sync_copy(ew_hbm.at[pl.ds(off, _DK)], ew_v)
        pltpu.sync_copy(ew_v, deg_sh.at[c_v], add=True)
        return carry

    lax.fori_loop(0, _DCH, chunk, 0)
    plsc.subcore_barrier()
    pltpu.sync_copy(deg_sh.at[pl.ds(sid * _DPT, _DPT)],
                    out_hbm.at[cid, pl.ds(sid * _DPT, _DPT)])


@functools.partial(
    pl.kernel,
    out_type=jax.ShapeDtypeStruct((_NC, _NP, _H), jnp.float32),
    mesh=_mesh,
    compiler_params=pltpu.CompilerParams(use_tc_tiling_on_sc=False,
                                        needs_layout_passes=False),
    scratch_types=[
        pltpu.VMEM_SHARED((_NP, _H), jnp.float32),
        pltpu.VMEM_SHARED((_NP, _H // 2), jnp.int32),
        pltpu.VMEM((_K, _H // 2), jnp.int32),
        pltpu.VMEM((_K, _H), jnp.float32),
        pltpu.VMEM((_K,), jnp.int32),
        pltpu.VMEM((_K,), jnp.int32),
        pltpu.VMEM((_K,), jnp.int32),
        pltpu.VMEM((_K,), jnp.int32),
        pltpu.VMEM((_K,), jnp.float32),
        pltpu.VMEM((_K,), jnp.float32),
        pltpu.SemaphoreType.DMA,
        pltpu.SemaphoreType.DMA,
        pltpu.SemaphoreType.DMA,
    ],
)
def _sc_message(gp_hbm, r_hbm, c_hbm, ew_hbm, out_hbm,
                acc, g_sh, gb, stage, r0, r1, c0, c1, w0, w1,
                gsem, isem0, isem1):
    cid = lax.axis_index("c")
    sid = lax.axis_index("s")
    wid = sid * _NC + cid
    r_b = (r0, r1)
    c_b = (c0, c1)
    w_b = (w0, w1)
    isem_b = (isem0, isem1)

    def zrow(i, carry):
        for q in range(_H // 16):
            stage[i, pl.ds(q * 16, 16)] = jnp.zeros((16,), jnp.float32)
        return carry

    lax.fori_loop(0, _K, zrow, 0)
    pltpu.sync_copy(gp_hbm.at[pl.ds(sid * _RPT, _RPT)],
                    g_sh.at[pl.ds(sid * _RPT, _RPT)])
    for t in range(13):
        pltpu.sync_copy(stage,
                        acc.at[pl.ds(sid * _RPT + t * _K, _K)])
    pltpu.sync_copy(stage.at[pl.ds(0, _RPT - 13 * _K)],
                    acc.at[pl.ds(sid * _RPT + 13 * _K, _RPT - 13 * _K)])
    plsc.subcore_barrier()

    base = wid * _EPT

    def idx_descs(i, b):
        off = base + i * _K
        return (
            pltpu.make_async_copy(r_hbm.at[pl.ds(off, _K)], r_b[b], isem_b[b]),
            pltpu.make_async_copy(c_hbm.at[pl.ds(off, _K)], c_b[b], isem_b[b]),
            pltpu.make_async_copy(ew_hbm.at[pl.ds(off, _K)], w_b[b],
                                  isem_b[b]),
        )

    for d in idx_descs(0, 0):
        d.start()
    for d in idx_descs(0, 0):
        d.wait()

    def chunk2(io, carry):
        for b in range(2):
            i = io * 2 + b
            nb = 1 - b
            pltpu.async_copy(g_sh.at[r_b[b]], gb, gsem).wait()

            @pl.when(i + 1 < _NCHUNK)
            def _next_idx():
                for d in idx_descs(i + 1, nb):
                    d.start()

            def scale16(jo, inner):
                j0 = jo * 16
                wv = w_b[b][pl.ds(j0, 16)]
                for jj in range(16):
                    w = jnp.full((16,), wv[jj], jnp.float32)
                    for q in range(_H // 32):
                        u = gb[j0 + jj, pl.ds(q * 16, 16)]
                        lo = plsc.bitcast(u << 16, jnp.float32)
                        hi = plsc.bitcast(
                            u & jnp.int32(-65536), jnp.float32)
                        stage[j0 + jj, pl.ds(q * 32, 16)] = lo * w
                        stage[j0 + jj, pl.ds(q * 32 + 16, 16)] = hi * w
                return inner

            lax.fori_loop(0, _K // 16, scale16, 0)

            pltpu.sync_copy(stage, acc.at[c_b[b]], add=True)

            @pl.when(i + 1 < _NCHUNK)
            def _land_idx():
                for d in idx_descs(i + 1, nb):
                    d.wait()
        return carry

    lax.fori_loop(0, _NCHUNK // 2, chunk2, 0)
    plsc.subcore_barrier()
    for t in range(13):
        o = sid * _RPT + t * _K
        pltpu.sync_copy(acc.at[pl.ds(o, _K)], out_hbm.at[cid, pl.ds(o, _K)])
    o = sid * _RPT + 13 * _K
    pltpu.sync_copy(acc.at[pl.ds(o, _RPT - 13 * _K)],
                    out_hbm.at[cid, pl.ds(o, _RPT - 13 * _K)])


def _tc1_body(d0_ref, d1_ref, x_ref, w1_ref, dis_ref, g1_ref):
    deg = d0_ref[...] + d1_ref[...] + 1.0
    dis = jnp.where(deg > 0.0, lax.rsqrt(deg), 0.0)
    dis_ref[...] = dis
    h = jnp.dot(x_ref[...], w1_ref[...], preferred_element_type=jnp.float32)
    g1_ref[...] = h * dis


def _tc2_body(a0_ref, a1_ref, g1_ref, dis_ref, b1_ref, w2_ref,
              h1_ref, g2_ref):
    dis = dis_ref[...]
    pre = (a0_ref[...] + a1_ref[...] + g1_ref[...]) * dis + b1_ref[...]
    h1 = jnp.maximum(pre, 0.0)
    h1_ref[...] = h1
    g2_ref[...] = jnp.dot(h1, w2_ref[...],
                          preferred_element_type=jnp.float32) * dis


def _tc3_body(a0_ref, a1_ref, g2_ref, dis_ref, b2_ref, h2_ref):
    pre = ((a0_ref[...] + a1_ref[...] + g2_ref[...]) * dis_ref[...]
           + b2_ref[...])
    h2_ref[...] = jnp.maximum(pre, 0.0)


def _row_blk(shape_cols):
    return pl.BlockSpec((_ROWBLK, shape_cols), lambda i: (i, 0))


def _full_blk(rows, cols):
    return pl.BlockSpec((rows, cols), lambda i: (0, 0))


_tc1 = pl.pallas_call(
    _tc1_body,
    grid=(_GRID,),
    in_specs=[
        _row_blk(1), _row_blk(1), _row_blk(_D), _full_blk(_D, _H),
    ],
    out_specs=[_row_blk(1), _row_blk(_H)],
    out_shape=[
        jax.ShapeDtypeStruct((_N, 1), jnp.float32),
        jax.ShapeDtypeStruct((_N, _H), jnp.float32),
    ],
)

_tc2 = pl.pallas_call(
    _tc2_body,
    grid=(_GRID,),
    in_specs=[
        _row_blk(_H), _row_blk(_H), _row_blk(_H), _row_blk(1),
        _full_blk(1, _H), _full_blk(_H, _H),
    ],
    out_specs=[_row_blk(_H), _row_blk(_H)],
    out_shape=[
        jax.ShapeDtypeStruct((_N, _H), jnp.float32),
        jax.ShapeDtypeStruct((_N, _H), jnp.float32),
    ],
)

_tc3 = pl.pallas_call(
    _tc3_body,
    grid=(_GRID,),
    in_specs=[
        _row_blk(_H), _row_blk(_H), _row_blk(_H), _row_blk(1),
        _full_blk(1, _H),
    ],
    out_specs=_row_blk(_H),
    out_shape=jax.ShapeDtypeStruct((_N, _H), jnp.float32),
)


def _pack_bf16(g):
    ge = g.astype(jnp.bfloat16).reshape(_N, _H // 32, 2, 16)
    gt = ge.transpose(0, 1, 3, 2)
    gi = lax.bitcast_convert_type(gt, jnp.int32).reshape(_N, _H // 2)
    return jnp.concatenate(
        [gi, jnp.zeros((_NP - _N, _H // 2), jnp.int32)], axis=0)


@jax.jit
def kernel(x, edge_index, edge_weights, W1, b1, W2, b2):
    row = edge_index[0]
    col = edge_index[1]
    pad = _EPAD - _E
    row_p = jnp.concatenate([row, jnp.zeros((pad,), jnp.int32)])
    col_p = jnp.concatenate([col, jnp.zeros((pad,), jnp.int32)])
    ew_p = jnp.concatenate([edge_weights, jnp.zeros((pad,), jnp.float32)])

    deg_parts = _sc_degree(col_p, ew_p)
    d0 = deg_parts[0, :_N].reshape(_N, 1)
    d1 = deg_parts[1, :_N].reshape(_N, 1)

    dis, g1 = _tc1(d0, d1, x, W1)

    acc1 = _sc_message(_pack_bf16(g1), row_p, col_p, ew_p)
    h1, g2 = _tc2(acc1[0, :_N], acc1[1, :_N], g1, dis,
                  b1.reshape(1, _H), W2)

    acc2 = _sc_message(_pack_bf16(g2), row_p, col_p, ew_p)
    h2 = _tc3(acc2[0, :_N], acc2[1, :_N], g2, dis, b2.reshape(1, _H))

    return jnp.concatenate([h1, h2], axis=-1)


# speedup vs baseline: 2.2590x; 1.0003x over previous
import functools

import jax
import jax.numpy as jnp
from jax import lax
from jax.experimental import pallas as pl
from jax.experimental.pallas import tpu as pltpu
from jax.experimental.pallas import tpu_sc as plsc

_N = 10000
_E = 320000
_D = 128
_H = 128

_NC = 2
_NS = 16
_NW = _NC * _NS

_K = 48
_NCHUNK = 224
_DK = 128
_DCH = 84
_EPT = _K * _NCHUNK
_EPAD = _EPT * _NW

_NP = 10112
_RPT = _NP // _NS
_ZROWS = 128

_DEGP = 10240
_DPT = _DEGP // _NS

_ROWBLK = 1000
_GRID = _N // _ROWBLK

_mesh = plsc.VectorSubcoreMesh(core_axis_name="c", subcore_axis_name="s")


@functools.partial(
    pl.kernel,
    out_type=jax.ShapeDtypeStruct((_NC, _DEGP), jnp.float32),
    mesh=_mesh,
    scratch_types=[
        pltpu.VMEM_SHARED((_DEGP,), jnp.float32),
        pltpu.VMEM((_DK,), jnp.int32),
        pltpu.VMEM((_DK,), jnp.float32),
        pltpu.VMEM((_DPT,), jnp.float32),
    ],
)
def _sc_degree(c_hbm, ew_hbm, out_hbm, deg_sh, c_v, ew_v, zbuf):
    cid = lax.axis_index("c")
    sid = lax.axis_index("s")
    wid = sid * _NC + cid

    for i in range(_DPT // 16):
        zbuf[pl.ds(i * 16, 16)] = jnp.zeros((16,), jnp.float32)
    pltpu.sync_copy(zbuf, deg_sh.at[pl.ds(sid * _DPT, _DPT)])
    plsc.subcore_barrier()

    base = wid * _EPT

    def chunk(i, carry):
        off = base + i * _DK
        pltpu.sync_copy(c_hbm.at[pl.ds(off, _DK)], c_v)
        pltpu.sync_copy(ew_hbm.at[pl.ds(off, _DK)], ew_v)
        pltpu.sync_copy(ew_v, deg_sh.at[c_v], add=True)
        return carry

    lax.fori_loop(0, _DCH, chunk, 0)
    plsc.subcore_barrier()
    pltpu.sync_copy(deg_sh.at[pl.ds(sid * _DPT, _DPT)],
                    out_hbm.at[cid, pl.ds(sid * _DPT, _DPT)])


@functools.partial(
    pl.kernel,
    out_type=jax.ShapeDtypeStruct((_NC, _NP, _H), jnp.float32),
    mesh=_mesh,
    compiler_params=pltpu.CompilerParams(use_tc_tiling_on_sc=False,
                                        needs_layout_passes=False),
    scratch_types=[
        pltpu.VMEM_SHARED((_NP, _H), jnp.float32),
        pltpu.VMEM_SHARED((_NP, _H // 2), jnp.int32),
        pltpu.VMEM((_K, _H // 2), jnp.int32),
        pltpu.VMEM((_K, _H), jnp.float32),
        pltpu.VMEM((3 * _K,), jnp.int32),
        pltpu.VMEM((3 * _K,), jnp.int32),
        pltpu.SemaphoreType.DMA,
        pltpu.SemaphoreType.DMA,
        pltpu.SemaphoreType.DMA,
        pltpu.SemaphoreType.DMA,
    ],
)
def _sc_message(gp_hbm, ed_hbm, out_hbm,
                acc, g_sh, gb, stage, eb0, eb1,
                gsem, ssem, isem0, isem1):
    cid = lax.axis_index("c")
    sid = lax.axis_index("s")
    wid = sid * _NC + cid
    eb = (eb0, eb1)
    isem_b = (isem0, isem1)

    def zrow(i, carry):
        for q in range(_H // 16):
            stage[i, pl.ds(q * 16, 16)] = jnp.zeros((16,), jnp.float32)
        return carry

    lax.fori_loop(0, _K, zrow, 0)
    pltpu.sync_copy(gp_hbm.at[pl.ds(sid * _RPT, _RPT)],
                    g_sh.at[pl.ds(sid * _RPT, _RPT)])
    for t in range(13):
        pltpu.sync_copy(stage,
                        acc.at[pl.ds(sid * _RPT + t * _K, _K)])
    pltpu.sync_copy(stage.at[pl.ds(0, _RPT - 13 * _K)],
                    acc.at[pl.ds(sid * _RPT + 13 * _K, _RPT - 13 * _K)])
    plsc.subcore_barrier()

    base = wid * _NCHUNK * 3 * _K

    def ed_desc(i, b):
        return pltpu.make_async_copy(
            ed_hbm.at[pl.ds(base + i * 3 * _K, 3 * _K)], eb[b], isem_b[b])

    d = ed_desc(0, 0)
    d.start()
    d.wait()

    def chunk2(io, carry):
        for b in range(2):
            i = io * 2 + b
            nb = 1 - b
            pltpu.async_copy(
                g_sh.at[eb[b].at[pl.ds(0, _K)]], gb, gsem).wait()

            @pl.when(i + 1 < _NCHUNK)
            def _next_ed():
                ed_desc(i + 1, nb).start()

            def scale16(jo, inner):
                j0 = jo * 16
                wv = plsc.bitcast(eb[b][pl.ds(2 * _K + j0, 16)], jnp.float32)
                for jj in range(16):
                    w = jnp.full((16,), wv[jj], jnp.float32)
                    for q in range(_H // 32):
                        u = gb[j0 + jj, pl.ds(q * 16, 16)]
                        lo = plsc.bitcast(u << 16, jnp.float32)
                        hi = plsc.bitcast(
                            u & jnp.int32(-65536), jnp.float32)
                        stage[j0 + jj, pl.ds(q * 32, 16)] = lo * w
                        stage[j0 + jj, pl.ds(q * 32 + 16, 16)] = hi * w
                return inner

            lax.fori_loop(0, _K // 16, scale16, 0)

            descs = []
            for sg in range(_K // 16):
                cvec = eb[b][pl.ds(_K + sg * 16, 16)]
                descs.append(pltpu.async_copy(
                    stage.at[pl.ds(sg * 16, 16)], acc.at[cvec], ssem,
                    add=True))
            for d_ in descs:
                d_.wait()

            @pl.when(i + 1 < _NCHUNK)
            def _land_ed():
                ed_desc(i + 1, nb).wait()
        return carry

    lax.fori_loop(0, _NCHUNK // 2, chunk2, 0)
    plsc.subcore_barrier()
    for t in range(13):
        o = sid * _RPT + t * _K
        pltpu.sync_copy(acc.at[pl.ds(o, _K)], out_hbm.at[cid, pl.ds(o, _K)])
    o = sid * _RPT + 13 * _K
    pltpu.sync_copy(acc.at[pl.ds(o, _RPT - 13 * _K)],
                    out_hbm.at[cid, pl.ds(o, _RPT - 13 * _K)])


def _tc1_body(d0_ref, d1_ref, x_ref, w1_ref, dis_ref, g1_ref):
    deg = d0_ref[...] + d1_ref[...] + 1.0
    dis = jnp.where(deg > 0.0, lax.rsqrt(deg), 0.0)
    dis_ref[...] = dis
    h = jnp.dot(x_ref[...], w1_ref[...], preferred_element_type=jnp.float32)
    g1_ref[...] = h * dis


def _tc2_body(a0_ref, a1_ref, g1_ref, dis_ref, b1_ref, w2_ref,
              h1_ref, g2_ref):
    dis = dis_ref[...]
    pre = (a0_ref[...] + a1_ref[...] + g1_ref[...]) * dis + b1_ref[...]
    h1 = jnp.maximum(pre, 0.0)
    h1_ref[...] = h1
    g2_ref[...] = jnp.dot(h1, w2_ref[...],
                          preferred_element_type=jnp.float32) * dis


def _tc3_body(a0_ref, a1_ref, g2_ref, dis_ref, b2_ref, h2_ref):
    pre = ((a0_ref[...] + a1_ref[...] + g2_ref[...]) * dis_ref[...]
           + b2_ref[...])
    h2_ref[...] = jnp.maximum(pre, 0.0)


def _row_blk(shape_cols):
    return pl.BlockSpec((_ROWBLK, shape_cols), lambda i: (i, 0))


def _full_blk(rows, cols):
    return pl.BlockSpec((rows, cols), lambda i: (0, 0))


_tc1 = pl.pallas_call(
    _tc1_body,
    grid=(_GRID,),
    in_specs=[
        _row_blk(1), _row_blk(1), _row_blk(_D), _full_blk(_D, _H),
    ],
    out_specs=[_row_blk(1), _row_blk(_H)],
    out_shape=[
        jax.ShapeDtypeStruct((_N, 1), jnp.float32),
        jax.ShapeDtypeStruct((_N, _H), jnp.float32),
    ],
)

_tc2 = pl.pallas_call(
    _tc2_body,
    grid=(_GRID,),
    in_specs=[
        _row_blk(_H), _row_blk(_H), _row_blk(_H), _row_blk(1),
        _full_blk(1, _H), _full_blk(_H, _H),
    ],
    out_specs=[_row_blk(_H), _row_blk(_H)],
    out_shape=[
        jax.ShapeDtypeStruct((_N, _H), jnp.float32),
        jax.ShapeDtypeStruct((_N, _H), jnp.float32),
    ],
)

_tc3 = pl.pallas_call(
    _tc3_body,
    grid=(_GRID,),
    in_specs=[
        _row_blk(_H), _row_blk(_H), _row_blk(_H), _row_blk(1),
        _full_blk(1, _H),
    ],
    out_specs=_row_blk(_H),
    out_shape=jax.ShapeDtypeStruct((_N, _H), jnp.float32),
)


def _pack_bf16(g):
    ge = g.astype(jnp.bfloat16).reshape(_N, _H // 32, 2, 16)
    gt = ge.transpose(0, 1, 3, 2)
    gi = lax.bitcast_convert_type(gt, jnp.int32).reshape(_N, _H // 2)
    return jnp.concatenate(
        [gi, jnp.zeros((_NP - _N, _H // 2), jnp.int32)], axis=0)


@jax.jit
def kernel(x, edge_index, edge_weights, W1, b1, W2, b2):
    row = edge_index[0]
    col = edge_index[1]
    pad = _EPAD - _E
    row_p = jnp.concatenate([row, jnp.zeros((pad,), jnp.int32)])
    col_p = jnp.concatenate([col, jnp.zeros((pad,), jnp.int32)])
    ew_p = jnp.concatenate([edge_weights, jnp.zeros((pad,), jnp.float32)])

    deg_parts = _sc_degree(col_p, ew_p)
    d0 = deg_parts[0, :_N].reshape(_N, 1)
    d1 = deg_parts[1, :_N].reshape(_N, 1)

    dis, g1 = _tc1(d0, d1, x, W1)

    ew_i = lax.bitcast_convert_type(ew_p, jnp.int32)
    edata = jnp.stack([row_p.reshape(-1, _K), col_p.reshape(-1, _K),
                       ew_i.reshape(-1, _K)], axis=1).reshape(-1)

    acc1 = _sc_message(_pack_bf16(g1), edata)
    h1, g2 = _tc2(acc1[0, :_N], acc1[1, :_N], g1, dis,
                  b1.reshape(1, _H), W2)

    acc2 = _sc_message(_pack_bf16(g2), edata)
    h2 = _tc3(acc2[0, :_N], acc2[1, :_N], g2, dis, b2.reshape(1, _H))

    return jnp.concatenate([h1, h2], axis=-1)
